# Initial kernel scaffold; baseline (speedup 1.0000x reference)
import jax
import jax.numpy as jnp
from jax.experimental import pallas as pl


def kernel(target, context, negative_samples, embeddings, context_embeddings):
    raise NotImplementedError("write your pallas kernel here")


# speedup vs baseline: 3.8799x; 3.8799x over previous
import functools

import jax
import jax.numpy as jnp
from jax import lax
from jax.experimental import pallas as pl
from jax.experimental.pallas import tpu as pltpu
from jax.experimental.pallas import tpu_sc as plsc

VOCAB = 1000000
DIM = 64
B = 16384
K = 20

_info = plsc.get_sparse_core_info()
NC, NS = _info.num_cores, _info.num_subcores
NW = NC * NS
BPW = B // NW
CHUNK = 128


def _sc_gather_body(tgt_idx, ctx_idx, neg_idx, emb, cemb,
                    t_out, c_out, n_out, idx_v, rows_v, sem):
    wid = lax.axis_index("s") * NC + lax.axis_index("c")
    base = wid * BPW

    def gather_chunks(idx_hbm, table, out_hbm, nchunks, off_fn):
        def body(j, _):
            off = off_fn(j)
            pltpu.sync_copy(idx_hbm.at[pl.ds(off, CHUNK)], idx_v)
            pltpu.async_copy(table.at[idx_v], rows_v, sem).wait()
            pltpu.sync_copy(rows_v, out_hbm.at[pl.ds(off, CHUNK)])
            return 0

        lax.fori_loop(0, nchunks, body, 0)

    gather_chunks(tgt_idx, emb, t_out, BPW // CHUNK, lambda j: base + j * CHUNK)
    gather_chunks(ctx_idx, cemb, c_out, BPW // CHUNK, lambda j: base + j * CHUNK)

    cpk = BPW // CHUNK

    def neg_off(j):
        k = j // cpk
        r = j - k * cpk
        return k * B + base + r * CHUNK

    gather_chunks(neg_idx, cemb, n_out, K * cpk, neg_off)


@functools.partial(
    pl.kernel,
    out_type=(
        jax.ShapeDtypeStruct((B, DIM), jnp.float32),
        jax.ShapeDtypeStruct((K * B, DIM), jnp.float32),
        jax.ShapeDtypeStruct((B, DIM), jnp.float32),
    ),
    mesh=plsc.VectorSubcoreMesh(core_axis_name="c", subcore_axis_name="s"),
    scratch_types=[
        pltpu.VMEM((CHUNK,), jnp.int32),
        pltpu.VMEM((CHUNK, DIM), jnp.float32),
        pltpu.SemaphoreType.DMA,
    ],
    compiler_params=pltpu.CompilerParams(use_tc_tiling_on_sc=False),
)
def _sc_gather(tgt_idx, ctx_idx, neg_idx, emb, cemb,
               t_out, n_out, c_out, idx_v, rows_v, sem):
    _sc_gather_body(tgt_idx, ctx_idx, neg_idx, emb, cemb,
                    t_out, c_out, n_out, idx_v, rows_v, sem)


BB = 512


def _log_sigmoid(x):
    return jnp.where(x >= 0, -jnp.log1p(jnp.exp(-x)), x - jnp.log1p(jnp.exp(x)))


def _tc_loss_kernel(t_ref, c_ref, n_ref, out_ref):
    t = t_ref[...]
    c = c_ref[...]
    pos = jnp.sum(t * c, axis=1)
    total = jnp.sum(_log_sigmoid(pos))
    for k in range(K):
        nk = n_ref[k]
        d = jnp.sum(nk * t, axis=1)
        total = total + jnp.sum(_log_sigmoid(-d))

    @pl.when(pl.program_id(0) == 0)
    def _init():
        out_ref[...] = jnp.zeros_like(out_ref)

    out_ref[...] += jnp.reshape(total, (1, 1))


def _tc_loss(t_rows, c_rows, n_rows):
    return pl.pallas_call(
        _tc_loss_kernel,
        grid=(B // BB,),
        in_specs=[
            pl.BlockSpec((BB, DIM), lambda i: (i, 0)),
            pl.BlockSpec((BB, DIM), lambda i: (i, 0)),
            pl.BlockSpec((K, BB, DIM), lambda i: (0, i, 0)),
        ],
        out_specs=pl.BlockSpec((1, 1), lambda i: (0, 0)),
        out_shape=jax.ShapeDtypeStruct((1, 1), jnp.float32),
    )(t_rows, c_rows, n_rows)


def kernel(target, context, negative_samples, embeddings, context_embeddings):
    tgt = target.astype(jnp.int32)
    ctx = context.astype(jnp.int32)
    neg = negative_samples.astype(jnp.int32).T.reshape(-1)

    t_rows, n_rows, c_rows = _sc_gather(tgt, ctx, neg, embeddings,
                                        context_embeddings)
    acc = _tc_loss(t_rows, c_rows, n_rows.reshape(K, B, DIM))
    return -acc[0, 0] / B


# speedup vs baseline: 4.0444x; 1.0424x over previous
import functools

import jax
import jax.numpy as jnp
from jax import lax
from jax.experimental import pallas as pl
from jax.experimental.pallas import tpu as pltpu
from jax.experimental.pallas import tpu_sc as plsc

VOCAB = 1000000
DIM = 64
B = 16384
K = 20
NSLOT = K + 2

_info = plsc.get_sparse_core_info()
NC, NS = _info.num_cores, _info.num_subcores
NW = NC * NS
BPW = B // NW
G = 32
NG = BPW // G


def _sc_body(idx_all, emb, cemb, dots_out, idxs_v, buf, dots_v, sem0, sem1):
    wid = lax.axis_index("s") * NC + lax.axis_index("c")
    base = wid * BPW
    sems = (sem0, sem1)

    pltpu.sync_copy(idx_all.at[wid], idxs_v)

    def fire(g):
        p = g % 2
        descs = []
        for slot in range(NSLOT):
            table = emb if slot == 0 else cemb
            descs.append(
                pltpu.async_copy(table.at[idxs_v.at[g, slot]],
                                 buf.at[p, slot], sems[p]))
        return descs

    def compute(g):
        p = g % 2
        bufp = buf.at[p]
        for s in range(G // 16):
            rows = s * 16 + lax.iota(jnp.int32, 16)

            def d_body(d, carry):
                dd = jnp.full((16,), d, jnp.int32)
                tvec = plsc.load_gather(
                    bufp, [jnp.full((16,), 0, jnp.int32), rows, dd])
                cvec = plsc.load_gather(
                    bufp, [jnp.full((16,), 1, jnp.int32), rows, dd])
                out = [carry[0] + tvec * cvec]
                for k in range(K):
                    nvec = plsc.load_gather(
                        bufp, [jnp.full((16,), 2 + k, jnp.int32), rows, dd])
                    out.append(carry[1 + k] + tvec * nvec)
                return tuple(out)

            init = tuple(jnp.zeros((16,), jnp.float32)
                         for _ in range(K + 1))
            accs = lax.fori_loop(0, DIM, d_body, init)
            off = g * G + s * 16
            for j in range(K + 1):
                dots_v[j, pl.ds(off, 16)] = accs[j]

    descs = fire(0)
    for g in range(NG):
        nxt = fire(g + 1) if g + 1 < NG else []
        for d in descs:
            d.wait()
        compute(g)
        descs = nxt

    pltpu.sync_copy(dots_v, dots_out.at[:, pl.ds(base, BPW)])


@functools.partial(
    pl.kernel,
    out_type=jax.ShapeDtypeStruct((K + 1, B), jnp.float32),
    mesh=plsc.VectorSubcoreMesh(core_axis_name="c", subcore_axis_name="s"),
    scratch_types=[
        pltpu.VMEM((NG, NSLOT, G), jnp.int32),
        pltpu.VMEM((2, NSLOT, G, DIM), jnp.float32),
        pltpu.VMEM((K + 1, BPW), jnp.float32),
        pltpu.SemaphoreType.DMA,
        pltpu.SemaphoreType.DMA,
    ],
    compiler_params=pltpu.CompilerParams(use_tc_tiling_on_sc=False,
                                         needs_layout_passes=False),
)
def _sc_dots(idx_all, emb, cemb, dots_out, idxs_v, buf, dots_v, sem0, sem1):
    _sc_body(idx_all, emb, cemb, dots_out, idxs_v, buf, dots_v, sem0, sem1)


BB = 2048


def _log_sigmoid(x):
    return jnp.where(x >= 0, -jnp.log1p(jnp.exp(-x)), x - jnp.log1p(jnp.exp(x)))


def _tc_loss_kernel(d_ref, out_ref):
    x = d_ref[...]
    pos = x[0:1, :]
    neg = x[1:, :]
    total = jnp.sum(_log_sigmoid(pos)) + jnp.sum(_log_sigmoid(-neg))

    @pl.when(pl.program_id(0) == 0)
    def _init():
        out_ref[...] = jnp.zeros_like(out_ref)

    out_ref[...] += jnp.reshape(total, (1, 1))


def _tc_loss(dots):
    return pl.pallas_call(
        _tc_loss_kernel,
        grid=(B // BB,),
        in_specs=[pl.BlockSpec((K + 1, BB), lambda i: (0, i))],
        out_specs=pl.BlockSpec((1, 1), lambda i: (0, 0)),
        out_shape=jax.ShapeDtypeStruct((1, 1), jnp.float32),
    )(dots)


def kernel(target, context, negative_samples, embeddings, context_embeddings):
    tgt = target.astype(jnp.int32).reshape(NW, NG, 1, G)
    ctx = context.astype(jnp.int32).reshape(NW, NG, 1, G)
    neg = (negative_samples.astype(jnp.int32)
           .reshape(NW, NG, G, K).transpose(0, 1, 3, 2))
    idx_all = jnp.concatenate([tgt, ctx, neg], axis=2)

    dots = _sc_dots(idx_all, embeddings, context_embeddings)
    acc = _tc_loss(dots)
    return -acc[0, 0] / B
